# Initial kernel scaffold; baseline (speedup 1.0000x reference)
import jax
import jax.numpy as jnp
from jax.experimental import pallas as pl


def kernel(marker_data, time_data, mask_data, embedding, neighbor_list, neighbor_prob, W_te, b_te, W_emb, b_emb, W_ih, b_ih, W_hh, b_hh, W_tl, b_tl, W_mk, b_mk):
    raise NotImplementedError("write your pallas kernel here")


# speedup vs baseline: 2.3953x; 2.3953x over previous
import functools

import jax
import jax.numpy as jnp
from jax import lax
from jax.experimental import pallas as pl
from jax.experimental.pallas import tpu as pltpu
from jax.experimental.pallas import tpu_sc as plsc

_B, _S = 64, 32
_NSTEP = _S - 1
_SS = 20
_EM = 64
_D = 128
_PW = 608
_RW = 624
_EW = 608
_BETA = 0.1
_MAX_TIME = 50.0


def _i16():
    return lax.iota(jnp.int32, 16)


def _sf(v, lane):
    return jnp.sum(jnp.where(_i16() == lane, v, jnp.zeros_like(v)))


def _si(v, lane):
    m = jnp.where(_i16() == lane, v, jnp.zeros_like(v))
    return jnp.sum(m.astype(jnp.float32)).astype(jnp.int32)


def _bc(x):
    return jnp.broadcast_to(x, (16,))


def _hi4(buf):
    idx = jnp.where(_i16() < 4, 16 + _i16(), 0)
    taken = plsc.load_gather(buf, [idx])
    return jnp.where(_i16() < 4, taken, jnp.zeros_like(taken))


def _softmax20(lo, hi):
    tail = _i16() < 4
    neg = jnp.float32(-3.0e38)
    hi_m = jnp.where(tail, hi, neg)
    mx = jnp.maximum(jnp.max(lo), jnp.max(hi_m))
    ulo = jnp.exp(lo - mx)
    uhi = jnp.where(tail, jnp.exp(hi_m - mx), jnp.float32(0.0))
    sm = jnp.sum(ulo) + jnp.sum(uhi)
    return ulo / sm, uhi / sm


def _sc_sample(m0_pad, neighbor_list, neighbor_prob, embedding, w1, enoise):
    mesh = plsc.VectorSubcoreMesh(core_axis_name="c", subcore_axis_name="s")

    @functools.partial(
        pl.kernel,
        out_type=[
            jax.ShapeDtypeStruct((_B, 32), jnp.int32),
            jax.ShapeDtypeStruct((_B, 32), jnp.float32),
            jax.ShapeDtypeStruct((_B, 32), jnp.float32),
            jax.ShapeDtypeStruct((_B, _EM), jnp.float32),
        ],
        mesh=mesh,
        compiler_params=pltpu.CompilerParams(needs_layout_passes=False),
        scratch_types=[
            pltpu.VMEM((16,), jnp.int32),
            pltpu.VMEM((_EM,), jnp.float32),
            pltpu.VMEM((_SS,), jnp.int32),
            pltpu.VMEM((_SS,), jnp.int32),
            pltpu.VMEM((_SS,), jnp.int32),
            pltpu.VMEM((_SS,), jnp.float32),
            pltpu.VMEM((_SS,), jnp.float32),
            pltpu.VMEM((_SS,), jnp.float32),
            pltpu.VMEM((_EM,), jnp.float32),
            pltpu.VMEM((_EM,), jnp.float32),
            pltpu.VMEM((60, _EM), jnp.float32),
            pltpu.VMEM((_NSTEP * _EW,), jnp.float32),
            pltpu.VMEM((_NSTEP * _EW,), jnp.float32),
            pltpu.VMEM((_PW,), jnp.float32),
            pltpu.VMEM((_PW,), jnp.float32),
            pltpu.VMEM((_PW,), jnp.int32),
            pltpu.VMEM((_PW,), jnp.int32),
            pltpu.VMEM((_RW,), jnp.float32),
            pltpu.VMEM((_RW,), jnp.float32),
            pltpu.VMEM((32,), jnp.int32),
            pltpu.VMEM((32,), jnp.int32),
            pltpu.VMEM((32,), jnp.float32),
            pltpu.VMEM((32,), jnp.float32),
            pltpu.VMEM((32,), jnp.float32),
            pltpu.VMEM((32,), jnp.float32),
            pltpu.SemaphoreType.DMA,
            pltpu.SemaphoreType.DMA,
            pltpu.SemaphoreType.DMA,
        ],
    )
    def k(m0_hbm, nl_hbm, np_hbm, emb_hbm, w1_hbm, e_hbm,
          samp_hbm, tnp_hbm, tsp_hbm, mv0_hbm,
          m0buf, w1buf, nl_r0, nl_r1, nl_00, np_r0, np_r1, np_00,
          mv0b0, mv0b1, ebn, ebuf0, ebuf1,
          prob0, prob1, cand0, cand1, rec0, rec1,
          mk0, mk1, tnp0, tnp1, tsp0, tsp1,
          sem_e0, sem_e1, sem_st):
        i16 = _i16()
        tail = i16 < 4
        wid = lax.axis_index("c") * 16 + lax.axis_index("s")
        b0 = wid * 2
        b1 = b0 + 1
        z16i = jnp.zeros((16,), jnp.int32)
        z16f = jnp.zeros((16,), jnp.float32)

        d_e0 = pltpu.async_copy(e_hbm.at[b0], ebuf0, sem_e0)
        d_e1 = pltpu.async_copy(e_hbm.at[b1], ebuf1, sem_e1)
        pltpu.sync_copy(m0_hbm.at[wid], m0buf)
        pltpu.sync_copy(w1_hbm, w1buf)
        m0v = m0buf[...]
        m_0 = _si(m0v, 0)
        m_1 = _si(m0v, 1)

        d_nl0 = pltpu.async_copy(nl_hbm.at[m_0], nl_r0, sem_st)
        d_nl1 = pltpu.async_copy(nl_hbm.at[m_1], nl_r1, sem_st)
        d_nlz = pltpu.async_copy(nl_hbm.at[0], nl_00, sem_st)
        d_np0 = pltpu.async_copy(np_hbm.at[m_0], np_r0, sem_st)
        d_np1 = pltpu.async_copy(np_hbm.at[m_1], np_r1, sem_st)
        d_npz = pltpu.async_copy(np_hbm.at[0], np_00, sem_st)
        d_mv0 = pltpu.async_copy(emb_hbm.at[m_0], mv0b0, sem_st)
        d_mv1 = pltpu.async_copy(emb_hbm.at[m_1], mv0b1, sem_st)
        d_nl0.wait()
        d_nl1.wait()
        d_nlz.wait()

        def _vecs(buf):
            return buf[pl.ds(0, 16)], _hi4(buf)

        nl_sets = [_vecs(nl_r0), _vecs(nl_r1), _vecs(nl_00)]

        d_rows = []
        for set_i, (lo, hi) in enumerate(nl_sets):
            for s in range(_SS):
                mj = _si(lo, s) if s < 16 else _si(hi, s - 16)
                d_rows.append(pltpu.async_copy(
                    emb_hbm.at[mj], ebn.at[set_i * _SS + s], sem_st))

        d_np0.wait()
        d_np1.wait()
        d_npz.wait()
        np_sets = [_vecs(np_r0), _vecs(np_r1), _vecs(np_00)]

        w1c = [w1buf[pl.ds(16 * kc, 16)] for kc in range(4)]
        for d in d_rows:
            d.wait()

        def _set_scores(set_i):
            lo = z16f
            hi = z16f
            for s in range(_SS):
                acc = z16f
                for kc in range(4):
                    v = ebn[set_i * _SS + s, pl.ds(16 * kc, 16)]
                    acc = acc + v * w1c[kc]
                rs = jnp.sum(acc)
                if s < 16:
                    lo = jnp.where(i16 == s, rs, lo)
                else:
                    hi = jnp.where(i16 == s - 16, rs, hi)
            return lo, hi

        mp_sets = [_softmax20(*_set_scores(si_)) for si_ in range(3)]

        rows = (
            (prob0, cand0, rec0, ebuf0, mk0, tnp0, tsp0, m_0,
             mp_sets[0], nl_sets[0], np_sets[0]),
            (prob1, cand1, rec1, ebuf1, mk1, tnp1, tsp1, m_1,
             mp_sets[1], nl_sets[1], np_sets[1]),
        )
        mp0_lo, mp0_hi = mp_sets[2]
        nl0_lo, nl0_hi = nl_sets[2]
        np0_lo, np0_hi = np_sets[2]
        one0 = jnp.where(i16 == 0, jnp.float32(1.0), jnp.float32(0.0))
        for (prob, cand, rec, _eb, mk, tnp, tsp, m_r, _mp, _nl, _np) in rows:
            def zf(kk, _, _prob=prob):
                _prob[pl.ds(kk * 16, 16)] = z16f
                return 0
            lax.fori_loop(0, _PW // 16, zf, 0)
            prob[pl.ds(0, 16)] = one0
            cand[pl.ds(0, 16)] = jnp.where(i16 == 0, _bc(m_r), z16i)
            rec[pl.ds(0, 16)] = one0
            tnp[pl.ds(0, 16)] = one0
            tsp[pl.ds(0, 16)] = one0

        d_e0.wait()
        d_e1.wait()

        def step(idx, carry):
            cs = list(carry)
            is0 = idx == 0
            wpre = 1 + 19 * idx
            rpre = 1 + 20 * idx
            wpost = 20 + 19 * idx
            nch = (wpost + 15) // 16
            ebase = idx * _EW
            for r, (prob, cand, rec, eb, mk, tnp, tsp, _m,
                    mp_r, nl_r, np_r) in enumerate(rows):
                c = cs[r]
                mp_lo = jnp.where(is0, mp_r[0], mp0_lo)
                mp_hi = jnp.where(is0, mp_r[1], mp0_hi)
                nl_lo = jnp.where(is0, nl_r[0], nl0_lo)
                nl_hi = jnp.where(is0, nl_r[1], nl0_hi)
                np_lo = jnp.where(is0, np_r[0], np0_lo)
                np_hi = jnp.where(is0, np_r[1], np0_hi)

                cvec = _bc(c)
                cp = plsc.load_gather(prob, [cvec])
                att_lo = cp * mp_lo
                att_hi = cp * mp_hi
                plsc.store_scatter(prob, [cvec], att_lo, mask=i16 == 0)
                plsc.store_scatter(prob, [(wpre - 1) + i16], att_lo, mask=i16 >= 1)
                plsc.store_scatter(prob, [wpre + 15 + i16], att_hi, mask=tail)
                plsc.store_scatter(cand, [(wpre - 1) + i16], nl_lo, mask=i16 >= 1)
                plsc.store_scatter(cand, [wpre + 15 + i16], nl_hi, mask=tail)
                plsc.store_scatter(rec, [rpre + i16], np_lo)
                plsc.store_scatter(rec, [rpre + 16 + i16], np_hi, mask=tail)

                def chunk(kk, cr, _prob=prob, _eb=eb, _ebase=ebase):
                    vmax, vci = cr
                    p = _prob[pl.ds(kk * 16, 16)]
                    e = _eb[pl.ds(_ebase + kk * 16, 16)]
                    sc = (p + jnp.float32(1e-20)) * e
                    upd = sc > vmax
                    return (jnp.where(upd, sc, vmax), jnp.where(upd, _bc(kk), vci))

                vmax, vci = lax.fori_loop(
                    0, nch, chunk,
                    (jnp.full((16,), -1.0, jnp.float32), z16i))
                gm = jnp.max(vmax)
                tie = vmax == gm
                bigf = jnp.float32(1e9)
                vci_f = vci.astype(jnp.float32)
                cmin_f = jnp.min(jnp.where(tie, vci_f, bigf))
                cmin = cmin_f.astype(jnp.int32)
                lmin_f = jnp.min(jnp.where(tie & (vci_f == cmin_f),
                                           i16.astype(jnp.float32), bigf))
                c_new = cmin * 16 + lmin_f.astype(jnp.int32)

                cnv = _bc(c_new)
                m_new = plsc.load_gather(cand, [cnv])
                np_sel = plsc.load_gather(rec, [cnv])
                sp_sel = plsc.load_gather(prob, [cnv])
                plsc.store_scatter(mk, [_bc(idx)], m_new, mask=i16 == 0)
                plsc.store_scatter(tnp, [_bc(idx + 1)], np_sel, mask=i16 == 0)
                plsc.store_scatter(tsp, [_bc(idx + 1)], sp_sel, mask=i16 == 0)
                cs[r] = c_new
            return tuple(cs)

        lax.fori_loop(0, _NSTEP, step, (jnp.int32(0), jnp.int32(0)))

        pltpu.sync_copy(mk0, samp_hbm.at[b0])
        pltpu.sync_copy(mk1, samp_hbm.at[b1])
        pltpu.sync_copy(tnp0, tnp_hbm.at[b0])
        pltpu.sync_copy(tnp1, tnp_hbm.at[b1])
        pltpu.sync_copy(tsp0, tsp_hbm.at[b0])
        pltpu.sync_copy(tsp1, tsp_hbm.at[b1])
        d_mv0.wait()
        d_mv1.wait()
        pltpu.sync_copy(mv0b0, mv0_hbm.at[b0])
        pltpu.sync_copy(mv0b1, mv0_hbm.at[b1])

    return k(m0_pad, neighbor_list, neighbor_prob, embedding, w1, enoise)


def _rnn_body(mv0_ref, e0_ref, t0_ref, m0_ref, wte_ref, bte_ref, wemb_ref,
              bemb_ref, wih_ref, bih_ref, whh_ref, bhh_ref, wtl_ref, btl_ref,
              tout_ref, mout_ref):
    t = t0_ref[:]
    h = jnp.zeros((_B, _D), jnp.float32)
    tout_ref[:, 0:1] = t
    mout_ref[:, 0:1] = m0_ref[:]
    e0 = e0_ref[0:1, :]
    for idx in range(_NSTEP):
        te = t * wte_ref[:] + bte_ref[:]
        if idx == 0:
            mv = mv0_ref[:]
        else:
            mv = jnp.broadcast_to(e0, (_B, _EM))
        nv = mv + _BETA * te
        x = jnp.dot(nv, wemb_ref[:], preferred_element_type=jnp.float32) + bemb_ref[:]
        x = jnp.where(x >= 0, x, 0.01 * x)
        h = jnp.tanh(
            jnp.dot(x, wih_ref[:], preferred_element_type=jnp.float32) + bih_ref[:]
            + jnp.dot(h, whh_ref[:], preferred_element_type=jnp.float32) + bhh_ref[:])
        d = jnp.dot(h, wtl_ref[:], preferred_element_type=jnp.float32) + btl_ref[:]
        t = t + jax.nn.softplus(d)
        tout_ref[:, idx + 1:idx + 2] = t
        mout_ref[:, idx + 1:idx + 2] = (t < _MAX_TIME).astype(jnp.float32)


def _rnn_tc(mv0, e0p, t0, m0col, wteT, bte, wembT, bemb, wihT, bih, whhT, bhh,
            wtlT, btl):
    return pl.pallas_call(
        _rnn_body,
        out_shape=[
            jax.ShapeDtypeStruct((_B, _S), jnp.float32),
            jax.ShapeDtypeStruct((_B, _S), jnp.float32),
        ],
    )(mv0, e0p, t0, m0col, wteT, bte, wembT, bemb, wihT, bih, whhT, bhh,
      wtlT, btl)


def _build_noise():
    skey = jax.random.key(1234)
    chunks = []
    for idx in range(_NSTEP):
        skey, sub = jax.random.split(skey)
        w = 20 + 19 * idx
        g = jax.random.gumbel(sub, (_B, w), jnp.float32)
        chunks.append(jnp.pad(jnp.exp(g), ((0, 0), (0, _EW - w))))
    return jnp.concatenate(chunks, axis=1)


def kernel(marker_data, time_data, mask_data, embedding, neighbor_list,
           neighbor_prob, W_te, b_te, W_emb, b_emb, W_ih, b_ih, W_hh, b_hh,
           W_tl, b_tl, W_mk, b_mk):
    m0 = marker_data[:, 0]
    m0_pad = jnp.pad(m0.reshape(32, 2), ((0, 0), (0, 14)))
    w1 = W_mk[0, :_EM]
    enoise = _build_noise()

    samp, tnp, tsp, mv0 = _sc_sample(
        m0_pad, neighbor_list, neighbor_prob, embedding, w1, enoise)

    time_res, mask_res = _rnn_tc(
        mv0, embedding[0:8], time_data[:, 0:1], mask_data[:, 0:1],
        W_te.T, b_te.reshape(1, _EM), W_emb.T, b_emb.reshape(1, _D),
        W_ih.T, b_ih.reshape(1, _D), W_hh.T, b_hh.reshape(1, _D),
        W_tl.T, b_tl.reshape(1, 1))

    marker_res = jnp.concatenate(
        [m0[:, None], jnp.zeros((_B, _NSTEP), jnp.int32), samp[:, :_NSTEP]],
        axis=1)
    return marker_res, time_res, mask_res, tnp, tsp


# speedup vs baseline: 6.2003x; 2.5885x over previous
import functools

import jax
import jax.numpy as jnp
import numpy as np
from jax import lax
from jax.experimental import pallas as pl
from jax.experimental.pallas import tpu as pltpu
from jax.experimental.pallas import tpu_sc as plsc

_SUBKEYS = np.array([
    [2877103387, 1697627890], [2352926074, 781486348],
    [1364783093, 4258707102], [1164617931, 1262857679],
    [3941076018, 3692555071], [2567592742, 1058756020],
    [2657002275, 4097592973], [3643699556, 3651991828],
    [903135717, 2747713321], [3007679383, 4275016182],
    [3610803866, 3938808000], [359505608, 1586013358],
    [3325904541, 3076085021], [990484358, 1537612016],
    [1513569712, 3128416685], [3611625703, 369928495],
    [3576089873, 3230020688], [228666783, 2258640005],
    [3365185757, 127975632], [1838060680, 4267770500],
    [1087441671, 2327599826], [313851471, 3082817180],
    [311282495, 2409185743], [4201892011, 911741220],
    [1013584138, 1159024862], [1550543751, 3059863652],
    [572474846, 2456354032], [2538718959, 369441366],
    [1529156089, 860003289], [1332119126, 642479970],
    [2630912896, 1592935602]], dtype=np.uint32)

_B, _S = 64, 32
_NSTEP = _S - 1
_SS = 20
_EM = 64
_D = 128
_PW = 608
_RW = 624
_EW = 608
_BETA = 0.1
_MAX_TIME = 50.0


def _i16():
    return lax.iota(jnp.int32, 16)


def _sf(v, lane):
    return jnp.sum(jnp.where(_i16() == lane, v, jnp.zeros_like(v)))


def _si(v, lane):
    m = jnp.where(_i16() == lane, v, jnp.zeros_like(v))
    return jnp.sum(m.astype(jnp.float32)).astype(jnp.int32)


def _bc(x):
    return jnp.broadcast_to(x, (16,))


def _hi4(buf):
    idx = jnp.where(_i16() < 4, 16 + _i16(), 0)
    taken = plsc.load_gather(buf, [idx])
    return jnp.where(_i16() < 4, taken, jnp.zeros_like(taken))


def _softmax20(lo, hi):
    tail = _i16() < 4
    neg = jnp.float32(-3.0e38)
    hi_m = jnp.where(tail, hi, neg)
    mx = jnp.maximum(jnp.max(lo), jnp.max(hi_m))
    ulo = jnp.exp(lo - mx)
    uhi = jnp.where(tail, jnp.exp(hi_m - mx), jnp.float32(0.0))
    sm = jnp.sum(ulo) + jnp.sum(uhi)
    return ulo / sm, uhi / sm


def _sc_sample(m0_pad, neighbor_list, neighbor_prob, embedding, w1, enoise):
    mesh = plsc.VectorSubcoreMesh(core_axis_name="c", subcore_axis_name="s")

    @functools.partial(
        pl.kernel,
        out_type=[
            jax.ShapeDtypeStruct((_B, 32), jnp.int32),
            jax.ShapeDtypeStruct((_B, 32), jnp.float32),
            jax.ShapeDtypeStruct((_B, 32), jnp.float32),
            jax.ShapeDtypeStruct((_B, _EM), jnp.float32),
        ],
        mesh=mesh,
        compiler_params=pltpu.CompilerParams(needs_layout_passes=False),
        scratch_types=[
            pltpu.VMEM((16,), jnp.int32),
            pltpu.VMEM((_EM,), jnp.float32),
            pltpu.VMEM((_SS,), jnp.int32),
            pltpu.VMEM((_SS,), jnp.int32),
            pltpu.VMEM((_SS,), jnp.int32),
            pltpu.VMEM((_SS,), jnp.float32),
            pltpu.VMEM((_SS,), jnp.float32),
            pltpu.VMEM((_SS,), jnp.float32),
            pltpu.VMEM((_EM,), jnp.float32),
            pltpu.VMEM((_EM,), jnp.float32),
            pltpu.VMEM((60, _EM), jnp.float32),
            pltpu.VMEM((_NSTEP * _EW,), jnp.float32),
            pltpu.VMEM((_NSTEP * _EW,), jnp.float32),
            pltpu.VMEM((_PW,), jnp.float32),
            pltpu.VMEM((_PW,), jnp.float32),
            pltpu.VMEM((_PW,), jnp.int32),
            pltpu.VMEM((_PW,), jnp.int32),
            pltpu.VMEM((_RW,), jnp.float32),
            pltpu.VMEM((_RW,), jnp.float32),
            pltpu.VMEM((32,), jnp.int32),
            pltpu.VMEM((32,), jnp.int32),
            pltpu.VMEM((32,), jnp.float32),
            pltpu.VMEM((32,), jnp.float32),
            pltpu.VMEM((32,), jnp.float32),
            pltpu.VMEM((32,), jnp.float32),
            pltpu.SemaphoreType.DMA,
            pltpu.SemaphoreType.DMA,
            pltpu.SemaphoreType.DMA,
        ],
    )
    def k(m0_hbm, nl_hbm, np_hbm, emb_hbm, w1_hbm, e_hbm,
          samp_hbm, tnp_hbm, tsp_hbm, mv0_hbm,
          m0buf, w1buf, nl_r0, nl_r1, nl_00, np_r0, np_r1, np_00,
          mv0b0, mv0b1, ebn, ebuf0, ebuf1,
          prob0, prob1, cand0, cand1, rec0, rec1,
          mk0, mk1, tnp0, tnp1, tsp0, tsp1,
          sem_e0, sem_e1, sem_st):
        i16 = _i16()
        tail = i16 < 4
        wid = lax.axis_index("c") * 16 + lax.axis_index("s")
        b0 = wid * 2
        b1 = b0 + 1
        z16i = jnp.zeros((16,), jnp.int32)
        z16f = jnp.zeros((16,), jnp.float32)

        d_e0 = pltpu.async_copy(e_hbm.at[b0], ebuf0, sem_e0)
        d_e1 = pltpu.async_copy(e_hbm.at[b1], ebuf1, sem_e1)
        pltpu.sync_copy(m0_hbm.at[wid], m0buf)
        pltpu.sync_copy(w1_hbm, w1buf)
        m0v = m0buf[...]
        m_0 = _si(m0v, 0)
        m_1 = _si(m0v, 1)

        d_nl0 = pltpu.async_copy(nl_hbm.at[m_0], nl_r0, sem_st)
        d_nl1 = pltpu.async_copy(nl_hbm.at[m_1], nl_r1, sem_st)
        d_nlz = pltpu.async_copy(nl_hbm.at[0], nl_00, sem_st)
        d_np0 = pltpu.async_copy(np_hbm.at[m_0], np_r0, sem_st)
        d_np1 = pltpu.async_copy(np_hbm.at[m_1], np_r1, sem_st)
        d_npz = pltpu.async_copy(np_hbm.at[0], np_00, sem_st)
        d_mv0 = pltpu.async_copy(emb_hbm.at[m_0], mv0b0, sem_st)
        d_mv1 = pltpu.async_copy(emb_hbm.at[m_1], mv0b1, sem_st)
        d_nl0.wait()
        d_nl1.wait()
        d_nlz.wait()

        def _vecs(buf):
            return buf[pl.ds(0, 16)], _hi4(buf)

        nl_sets = [_vecs(nl_r0), _vecs(nl_r1), _vecs(nl_00)]

        d_rows = []
        for set_i, (lo, hi) in enumerate(nl_sets):
            for s in range(_SS):
                mj = _si(lo, s) if s < 16 else _si(hi, s - 16)
                d_rows.append(pltpu.async_copy(
                    emb_hbm.at[mj], ebn.at[set_i * _SS + s], sem_st))

        d_np0.wait()
        d_np1.wait()
        d_npz.wait()
        np_sets = [_vecs(np_r0), _vecs(np_r1), _vecs(np_00)]

        w1c = [w1buf[pl.ds(16 * kc, 16)] for kc in range(4)]
        for d in d_rows:
            d.wait()

        def _set_scores(set_i):
            lo = z16f
            hi = z16f
            for s in range(_SS):
                acc = z16f
                for kc in range(4):
                    v = ebn[set_i * _SS + s, pl.ds(16 * kc, 16)]
                    acc = acc + v * w1c[kc]
                rs = jnp.sum(acc)
                if s < 16:
                    lo = jnp.where(i16 == s, rs, lo)
                else:
                    hi = jnp.where(i16 == s - 16, rs, hi)
            return lo, hi

        mp_sets = [_softmax20(*_set_scores(si_)) for si_ in range(3)]

        rows = (
            (prob0, cand0, rec0, ebuf0, mk0, tnp0, tsp0, m_0,
             mp_sets[0], nl_sets[0], np_sets[0]),
            (prob1, cand1, rec1, ebuf1, mk1, tnp1, tsp1, m_1,
             mp_sets[1], nl_sets[1], np_sets[1]),
        )
        mp0_lo, mp0_hi = mp_sets[2]
        nl0_lo, nl0_hi = nl_sets[2]
        np0_lo, np0_hi = np_sets[2]
        one0 = jnp.where(i16 == 0, jnp.float32(1.0), jnp.float32(0.0))
        for (prob, cand, rec, _eb, mk, tnp, tsp, m_r, _mp, _nl, _np) in rows:
            def zf(kk, _, _prob=prob):
                _prob[pl.ds(kk * 16, 16)] = z16f
                return 0
            lax.fori_loop(0, _PW // 16, zf, 0)
            prob[pl.ds(0, 16)] = one0
            cand[pl.ds(0, 16)] = jnp.where(i16 == 0, _bc(m_r), z16i)
            rec[pl.ds(0, 16)] = one0
            tnp[pl.ds(0, 16)] = one0
            tsp[pl.ds(0, 16)] = one0

        d_e0.wait()
        d_e1.wait()

        def step(idx, carry):
            cs = list(carry)
            is0 = idx == 0
            wpre = 1 + 19 * idx
            rpre = 1 + 20 * idx
            wpost = 20 + 19 * idx
            nch = (wpost + 15) // 16
            ebase = idx * _EW
            for r, (prob, cand, rec, eb, mk, tnp, tsp, _m,
                    mp_r, nl_r, np_r) in enumerate(rows):
                c = cs[r]
                mp_lo = jnp.where(is0, mp_r[0], mp0_lo)
                mp_hi = jnp.where(is0, mp_r[1], mp0_hi)
                nl_lo = jnp.where(is0, nl_r[0], nl0_lo)
                nl_hi = jnp.where(is0, nl_r[1], nl0_hi)
                np_lo = jnp.where(is0, np_r[0], np0_lo)
                np_hi = jnp.where(is0, np_r[1], np0_hi)

                cvec = _bc(c)
                cp = plsc.load_gather(prob, [cvec])
                att_lo = cp * mp_lo
                att_hi = cp * mp_hi
                plsc.store_scatter(prob, [cvec], att_lo, mask=i16 == 0)
                plsc.store_scatter(prob, [(wpre - 1) + i16], att_lo, mask=i16 >= 1)
                plsc.store_scatter(prob, [wpre + 15 + i16], att_hi, mask=tail)
                plsc.store_scatter(cand, [(wpre - 1) + i16], nl_lo, mask=i16 >= 1)
                plsc.store_scatter(cand, [wpre + 15 + i16], nl_hi, mask=tail)
                plsc.store_scatter(rec, [rpre + i16], np_lo)
                plsc.store_scatter(rec, [rpre + 16 + i16], np_hi, mask=tail)

                def chunk(kk, cr, _prob=prob, _eb=eb, _ebase=ebase):
                    vmax, vci = cr
                    p = _prob[pl.ds(kk * 16, 16)]
                    e = _eb[pl.ds(_ebase + kk * 16, 16)]
                    sc = (p + jnp.float32(1e-20)) * e
                    upd = sc > vmax
                    return (jnp.where(upd, sc, vmax), jnp.where(upd, _bc(kk), vci))

                vmax, vci = lax.fori_loop(
                    0, nch, chunk,
                    (jnp.full((16,), -1.0, jnp.float32), z16i))
                gm = jnp.max(vmax)
                tie = vmax == gm
                bigf = jnp.float32(1e9)
                vci_f = vci.astype(jnp.float32)
                cmin_f = jnp.min(jnp.where(tie, vci_f, bigf))
                cmin = cmin_f.astype(jnp.int32)
                lmin_f = jnp.min(jnp.where(tie & (vci_f == cmin_f),
                                           i16.astype(jnp.float32), bigf))
                c_new = cmin * 16 + lmin_f.astype(jnp.int32)

                cnv = _bc(c_new)
                m_new = plsc.load_gather(cand, [cnv])
                np_sel = plsc.load_gather(rec, [cnv])
                sp_sel = plsc.load_gather(prob, [cnv])
                plsc.store_scatter(mk, [_bc(idx)], m_new, mask=i16 == 0)
                plsc.store_scatter(tnp, [_bc(idx + 1)], np_sel, mask=i16 == 0)
                plsc.store_scatter(tsp, [_bc(idx + 1)], sp_sel, mask=i16 == 0)
                cs[r] = c_new
            return tuple(cs)

        lax.fori_loop(0, _NSTEP, step, (jnp.int32(0), jnp.int32(0)))

        pltpu.sync_copy(mk0, samp_hbm.at[b0])
        pltpu.sync_copy(mk1, samp_hbm.at[b1])
        pltpu.sync_copy(tnp0, tnp_hbm.at[b0])
        pltpu.sync_copy(tnp1, tnp_hbm.at[b1])
        pltpu.sync_copy(tsp0, tsp_hbm.at[b0])
        pltpu.sync_copy(tsp1, tsp_hbm.at[b1])
        d_mv0.wait()
        d_mv1.wait()
        pltpu.sync_copy(mv0b0, mv0_hbm.at[b0])
        pltpu.sync_copy(mv0b1, mv0_hbm.at[b1])

    return k(m0_pad, neighbor_list, neighbor_prob, embedding, w1, enoise)


def _rnn_body(mv0_ref, e0_ref, t0_ref, m0_ref, wte_ref, bte_ref, wemb_ref,
              bemb_ref, wih_ref, bih_ref, whh_ref, bhh_ref, wtl_ref, btl_ref,
              tout_ref, mout_ref):
    t = t0_ref[:]
    h = jnp.zeros((_B, _D), jnp.float32)
    tout_ref[:, 0:1] = t
    mout_ref[:, 0:1] = m0_ref[:]
    e0 = e0_ref[0:1, :]
    for idx in range(_NSTEP):
        te = t * wte_ref[:] + bte_ref[:]
        if idx == 0:
            mv = mv0_ref[:]
        else:
            mv = jnp.broadcast_to(e0, (_B, _EM))
        nv = mv + _BETA * te
        x = jnp.dot(nv, wemb_ref[:], preferred_element_type=jnp.float32) + bemb_ref[:]
        x = jnp.where(x >= 0, x, 0.01 * x)
        h = jnp.tanh(
            jnp.dot(x, wih_ref[:], preferred_element_type=jnp.float32) + bih_ref[:]
            + jnp.dot(h, whh_ref[:], preferred_element_type=jnp.float32) + bhh_ref[:])
        d = jnp.dot(h, wtl_ref[:], preferred_element_type=jnp.float32) + btl_ref[:]
        t = t + jax.nn.softplus(d)
        tout_ref[:, idx + 1:idx + 2] = t
        mout_ref[:, idx + 1:idx + 2] = (t < _MAX_TIME).astype(jnp.float32)


def _rnn_tc(mv0, e0p, t0, m0col, wteT, bte, wembT, bemb, wihT, bih, whhT, bhh,
            wtlT, btl):
    return pl.pallas_call(
        _rnn_body,
        out_shape=[
            jax.ShapeDtypeStruct((_B, _S), jnp.float32),
            jax.ShapeDtypeStruct((_B, _S), jnp.float32),
        ],
    )(mv0, e0p, t0, m0col, wteT, bte, wembT, bemb, wihT, bih, whhT, bhh,
      wtlT, btl)


def _noise_body(out_ref):
    rots = (13, 15, 26, 6, 17, 29, 16, 24)
    jcol = lax.broadcasted_iota(jnp.uint32, (_B, _EW), 1)
    brow = lax.broadcasted_iota(jnp.uint32, (_B, _EW), 0)
    tiny = jnp.float32(np.float32(np.finfo(np.float32).tiny))
    one = jnp.float32(1.0)
    for idx in range(_NSTEP):
        w = 20 + 19 * idx
        k1 = np.uint32(_SUBKEYS[idx, 0])
        k2 = np.uint32(_SUBKEYS[idx, 1])
        ks = [k1, k2, np.uint32(k1 ^ k2 ^ np.uint32(0x1BD11BDA))]
        f = brow * np.uint32(w) + jcol
        x0 = jnp.full((_B, _EW), ks[0], jnp.uint32)
        x1 = f + ks[1]
        for i in range(5):
            for r in (rots[:4] if i % 2 == 0 else rots[4:]):
                x0 = x0 + x1
                x1 = (x1 << np.uint32(r)) | (x1 >> np.uint32(32 - r))
                x1 = x0 ^ x1
            x0 = x0 + ks[(i + 1) % 3]
            x1 = x1 + ks[(i + 2) % 3] + np.uint32(i + 1)
        bits = x0 ^ x1
        fb = (bits >> np.uint32(9)) | np.uint32(0x3F800000)
        fl = lax.bitcast_convert_type(fb, jnp.float32) - one
        u = jnp.maximum(tiny, fl * (one - tiny) + tiny)
        e = jnp.exp(-jnp.log(-jnp.log(u)))
        out_ref[:, idx * _EW:(idx + 1) * _EW] = jnp.where(
            jcol < np.uint32(w), e, jnp.float32(0.0))


def _build_noise():
    return pl.pallas_call(
        _noise_body,
        out_shape=jax.ShapeDtypeStruct((_B, _NSTEP * _EW), jnp.float32),
    )()


def kernel(marker_data, time_data, mask_data, embedding, neighbor_list,
           neighbor_prob, W_te, b_te, W_emb, b_emb, W_ih, b_ih, W_hh, b_hh,
           W_tl, b_tl, W_mk, b_mk):
    m0 = marker_data[:, 0]
    m0_pad = jnp.pad(m0.reshape(32, 2), ((0, 0), (0, 14)))
    w1 = W_mk[0, :_EM]
    enoise = _build_noise()

    samp, tnp, tsp, mv0 = _sc_sample(
        m0_pad, neighbor_list, neighbor_prob, embedding, w1, enoise)

    time_res, mask_res = _rnn_tc(
        mv0, embedding[0:8], time_data[:, 0:1], mask_data[:, 0:1],
        W_te.T, b_te.reshape(1, _EM), W_emb.T, b_emb.reshape(1, _D),
        W_ih.T, b_ih.reshape(1, _D), W_hh.T, b_hh.reshape(1, _D),
        W_tl.T, b_tl.reshape(1, 1))

    marker_res = jnp.concatenate(
        [m0[:, None], jnp.zeros((_B, _NSTEP), jnp.int32), samp[:, :_NSTEP]],
        axis=1)
    return marker_res, time_res, mask_res, tnp, tsp
